# Initial kernel scaffold; baseline (speedup 1.0000x reference)
import jax
import jax.numpy as jnp
from jax.experimental import pallas as pl


def kernel(tokens, values, table, W_val, b_val, tok_g, tok_b, val_g, val_b, fin_g, fin_b):
    raise NotImplementedError("write your pallas kernel here")


# speedup vs baseline: 1.7653x; 1.7653x over previous
import functools

import jax
import jax.numpy as jnp
from jax import lax
from jax.experimental import pallas as pl
from jax.experimental.pallas import tpu as pltpu
from jax.experimental.pallas import tpu_sc as plsc

_EPS = 1e-5
_CW = 128


def _sc_gather(table, idx3):
    nw, nch, cw = idx3.shape
    d = table.shape[1]
    mesh = plsc.VectorSubcoreMesh(core_axis_name="c", subcore_axis_name="s")

    @functools.partial(
        pl.kernel,
        mesh=mesh,
        out_type=jax.ShapeDtypeStruct((nw * nch * cw, d), jnp.float32),
        scratch_types=[
            pltpu.VMEM((nch, cw), jnp.int32),
            pltpu.VMEM((2, cw, d), jnp.float32),
            pltpu.SemaphoreType.DMA,
            pltpu.SemaphoreType.DMA,
        ],
        compiler_params=pltpu.CompilerParams(use_tc_tiling_on_sc=False),
    )
    def k(table_hbm, idx_hbm, out_hbm, idx_v, rows_v, sem0, sem1):
        wid = lax.axis_index("s") * 2 + lax.axis_index("c")
        base = wid * (nch * cw)
        sems = (sem0, sem1)
        pltpu.sync_copy(idx_hbm.at[wid], idx_v)

        for b in range(2):
            pltpu.async_copy(table_hbm.at[idx_v.at[b]], rows_v.at[b], sems[b])

        def pair(i, _):
            for b in range(2):
                j = 2 * i + b
                pltpu.make_async_copy(
                    table_hbm.at[idx_v.at[j]], rows_v.at[b], sems[b]
                ).wait()
                pltpu.sync_copy(rows_v.at[b],
                                out_hbm.at[pl.ds(base + j * cw, cw)])
                pltpu.async_copy(table_hbm.at[idx_v.at[j + 2]], rows_v.at[b],
                                 sems[b])
            return 0

        lax.fori_loop(0, nch // 2 - 1, pair, 0)

        for b in range(2):
            j = nch - 2 + b
            pltpu.make_async_copy(
                table_hbm.at[idx_v.at[j]], rows_v.at[b], sems[b]
            ).wait()
            pltpu.sync_copy(rows_v.at[b], out_hbm.at[pl.ds(base + j * cw, cw)])

    return k(table, idx3)


def _ln(y, g, b):
    mu = jnp.mean(y, axis=-1, keepdims=True)
    c = y - mu
    var = jnp.mean(c * c, axis=-1, keepdims=True)
    return c * lax.rsqrt(var + _EPS) * g + b


def _tc_body(x_ref, v_ref, w_ref, bv_ref, tg_ref, tb_ref, vg_ref, vb_ref,
             fg_ref, fb_ref, o_ref):
    x = x_ref[...]
    v = v_ref[...]
    tok = _ln(x, tg_ref[...], tb_ref[...])
    ve = v * w_ref[...] + bv_ref[...]
    val = _ln(ve, vg_ref[...], vb_ref[...])
    o_ref[...] = _ln((tok + val) * 8.0, fg_ref[...], fb_ref[...])


def _tc_fused(gathered, vals2, w2, bv2, tg, tb, vg, vb, fg, fb, rows):
    bl, d = gathered.shape
    grid = (bl // rows,)
    wspec = pl.BlockSpec((1, d), lambda i: (0, 0))
    return pl.pallas_call(
        _tc_body,
        grid=grid,
        in_specs=[
            pl.BlockSpec((rows, d), lambda i: (i, 0)),
            pl.BlockSpec((rows, 1), lambda i: (i, 0)),
            wspec, wspec, wspec, wspec, wspec, wspec, wspec, wspec,
        ],
        out_specs=pl.BlockSpec((rows, d), lambda i: (i, 0)),
        out_shape=jax.ShapeDtypeStruct((bl, d), jnp.float32),
    )(gathered, vals2, w2, bv2, tg, tb, vg, vb, fg, fb)


def kernel(tokens, values, table, W_val, b_val, tok_g, tok_b, val_g, val_b,
           fin_g, fin_b):
    b, l = tokens.shape
    d = table.shape[1]
    bl = b * l
    nw = 32
    nch = bl // (nw * _CW)

    idx3 = tokens.reshape(nw, nch, _CW).astype(jnp.int32)
    gathered = _sc_gather(table, idx3)

    r2 = lambda a: a.reshape(1, d)
    out = _tc_fused(
        gathered, values.reshape(bl, 1),
        r2(W_val), r2(b_val), r2(tok_g), r2(tok_b), r2(val_g), r2(val_b),
        r2(fin_g), r2(fin_b), rows=1024,
    )
    return (out.reshape(b, l, d), tokens != 0)
